# baseline (speedup 1.0000x reference)
import functools

import jax
import jax.numpy as jnp
from jax import lax
from jax.experimental import pallas as pl
from jax.experimental.pallas import tpu as pltpu

_QPAD = 128
_A = 6


def _round_up(x, m):
    return (x + m - 1) // m * m


def _fwd_kernel(x_ref, w1_ref, w23_ref, wh_ref, b_ref, q_ref, *,
                tb, c1, c2, c3, two_h):
    f32 = jnp.float32
    cdt = w1_ref.dtype

    xv = x_ref[...]
    zt = []
    for t in range(4):
        rhs = w1_ref[t * 64:(t + 1) * 64, :]
        zt.append(jnp.dot(xv, rhs, preferred_element_type=f32)
                  .reshape(9, 9, tb, c1))
    z1 = (zt[0][0:8, 0:8] + zt[1][0:8, 1:9]
          + zt[2][1:9, 0:8] + zt[3][1:9, 1:9])
    a1 = jnp.maximum(z1 + b_ref[0:1, 0:c1], 0.0).astype(cdt)
    a1v = a1.reshape(4, 2, 4, 2, tb, c1)

    z2 = None
    for th in range(4):
        for tw in range(4):
            lhs = a1v[th // 2:th // 2 + 3, th % 2,
                      tw // 2:tw // 2 + 3, tw % 2].reshape(9 * tb, c1)
            t = th * 4 + tw
            rhs = w23_ref[t * c1:(t + 1) * c1, :]
            c = jnp.dot(lhs, rhs, preferred_element_type=f32)
            z2 = c if z2 is None else z2 + c
    x2 = jnp.maximum(z2 + b_ref[1:2, 0:c2], 0.0).astype(cdt)

    w3_off = 16 * c1
    zf = None
    for p in range(9):
        lhs = x2[p * tb:(p + 1) * tb, :]
        rhs = w23_ref[w3_off + p * c2:w3_off + (p + 1) * c2, :]
        c = jnp.dot(lhs, rhs, preferred_element_type=f32)
        zf = c if zf is None else zf + c
    feat = jnp.maximum(zf + b_ref[2:3, 0:c3], 0.0).astype(cdt)

    h = jnp.maximum(
        jnp.dot(feat, wh_ref[0:c3, 0:two_h], preferred_element_type=f32)
        + b_ref[3:4, 0:two_h], 0.0).astype(cdt)
    qpad = q_ref.shape[1]
    va = (jnp.dot(h, wh_ref[c3:c3 + two_h, 0:qpad],
                  preferred_element_type=f32)
          + b_ref[4:5, 0:qpad])

    value = va[:, 0:1]
    col = lax.broadcasted_iota(jnp.int32, va.shape, 1)
    adv_mask = (col >= 1) & (col < 1 + _A)
    adv_mean = jnp.sum(jnp.where(adv_mask, va, 0.0), axis=1,
                       keepdims=True) * (1.0 / _A)
    q_ref[...] = value + va - adv_mean


def kernel(w1, w23, whead, biases, state_nchw, *, batch_tile=128):
    cdt = w1.dtype
    B = state_nchw.shape[0]
    c1 = w1.shape[1]
    c2 = w23.shape[1]
    c3 = c2
    hidden = (whead.shape[0] - c3) // 2
    two_h = 2 * hidden

    tb = min(_round_up(B, 16), batch_tile)
    n_tiles = pl.cdiv(B, tb)
    G = n_tiles * tb

    w1s = (w1.reshape(2, 4, 2, 4, 4, c1)
             .transpose(0, 2, 4, 1, 3, 5).reshape(4 * 64, c1))

    x = state_nchw
    if G != B:
        x = jnp.pad(x, ((0, G - B), (0, 0), (0, 0), (0, 0)))
    xs = x.reshape(n_tiles, tb, 4, 9, 4, 9, 4)
    xs = (xs.transpose(0, 3, 5, 1, 2, 4, 6)
            .reshape(n_tiles, 81 * tb, 64).astype(cdt))

    body = functools.partial(_fwd_kernel, tb=tb, c1=c1, c2=c2, c3=c3,
                             two_h=two_h)
    out = pl.pallas_call(
        body,
        out_shape=jax.ShapeDtypeStruct((G, _QPAD), jnp.float32),
        grid=(n_tiles,),
        in_specs=[
            pl.BlockSpec((None, 81 * tb, 64), lambda b: (b, 0, 0)),
            pl.BlockSpec(w1s.shape, lambda b: (0, 0)),
            pl.BlockSpec(w23.shape, lambda b: (0, 0)),
            pl.BlockSpec(whead.shape, lambda b: (0, 0)),
            pl.BlockSpec(biases.shape, lambda b: (0, 0)),
        ],
        out_specs=pl.BlockSpec((tb, _QPAD), lambda b: (b, 0)),
        compiler_params=pltpu.CompilerParams(
            dimension_semantics=("parallel",),
            vmem_limit_bytes=48 * 1024 * 1024),
    )(xs, w1s, w23, whead, biases)

    return out[:B, 1:1 + _A]


# speedup vs baseline: 1.1506x; 1.1506x over previous
import functools

import jax
import jax.numpy as jnp
from jax import lax
from jax.experimental import pallas as pl
from jax.experimental.pallas import tpu as pltpu

_QPAD = 128
_A = 6


def _round_up(x, m):
    return (x + m - 1) // m * m


def _fwd_kernel(x_ref, w1_ref, w23_ref, wh_ref, b_ref, q_ref, *,
                tb, c1, c2, c3, two_h):
    f32 = jnp.float32
    cdt = w1_ref.dtype

    xv = x_ref[...]
    zt = []
    for t in range(4):
        rhs = w1_ref[t * 64:(t + 1) * 64, :]
        zt.append(lax.dot_general(xv, rhs, (((0,), (0,)), ((), ())),
                                  preferred_element_type=f32)
                  .reshape(9, 9, tb, c1))
    z1 = (zt[0][0:8, 0:8] + zt[1][0:8, 1:9]
          + zt[2][1:9, 0:8] + zt[3][1:9, 1:9])
    a1 = jnp.maximum(z1 + b_ref[0:1, 0:c1], 0.0).astype(cdt)
    a1v = a1.reshape(4, 2, 4, 2, tb, c1)

    z2 = None
    for th in range(4):
        for tw in range(4):
            lhs = a1v[th // 2:th // 2 + 3, th % 2,
                      tw // 2:tw // 2 + 3, tw % 2].reshape(9 * tb, c1)
            t = th * 4 + tw
            rhs = w23_ref[t * c1:(t + 1) * c1, :]
            c = jnp.dot(lhs, rhs, preferred_element_type=f32)
            z2 = c if z2 is None else z2 + c
    x2 = jnp.maximum(z2 + b_ref[1:2, 0:c2], 0.0).astype(cdt)

    w3_off = 16 * c1
    zf = None
    for p in range(9):
        lhs = x2[p * tb:(p + 1) * tb, :]
        rhs = w23_ref[w3_off + p * c2:w3_off + (p + 1) * c2, :]
        c = jnp.dot(lhs, rhs, preferred_element_type=f32)
        zf = c if zf is None else zf + c
    feat = jnp.maximum(zf + b_ref[2:3, 0:c3], 0.0).astype(cdt)

    h = jnp.maximum(
        jnp.dot(feat, wh_ref[0:c3, 0:two_h], preferred_element_type=f32)
        + b_ref[3:4, 0:two_h], 0.0).astype(cdt)
    qpad = q_ref.shape[1]
    va = (jnp.dot(h, wh_ref[c3:c3 + two_h, 0:qpad],
                  preferred_element_type=f32)
          + b_ref[4:5, 0:qpad])

    value = va[:, 0:1]
    col = lax.broadcasted_iota(jnp.int32, va.shape, 1)
    adv_mask = (col >= 1) & (col < 1 + _A)
    adv_mean = jnp.sum(jnp.where(adv_mask, va, 0.0), axis=1,
                       keepdims=True) * (1.0 / _A)
    q_ref[...] = value + va - adv_mean


def kernel(w1, w23, whead, biases, state_nchw, *, batch_tile=128):
    cdt = w1.dtype
    B = state_nchw.shape[0]
    c1 = w1.shape[1]
    c2 = w23.shape[1]
    c3 = c2
    hidden = (whead.shape[0] - c3) // 2
    two_h = 2 * hidden

    tb = min(_round_up(B, 16), batch_tile)
    n_tiles = pl.cdiv(B, tb)
    G = n_tiles * tb

    w1s = (w1.reshape(2, 4, 2, 4, 4, c1)
             .transpose(0, 2, 4, 1, 3, 5).reshape(4 * 64, c1))

    x = state_nchw
    if G != B:
        x = jnp.pad(x, ((0, G - B), (0, 0), (0, 0), (0, 0)))
    xs = x.reshape(n_tiles, tb, 4, 9, 4, 9, 4)
    xs = (xs.transpose(0, 2, 4, 6, 3, 5, 1)
            .reshape(n_tiles, 64, 81 * tb).astype(cdt))

    body = functools.partial(_fwd_kernel, tb=tb, c1=c1, c2=c2, c3=c3,
                             two_h=two_h)
    out = pl.pallas_call(
        body,
        out_shape=jax.ShapeDtypeStruct((G, _QPAD), jnp.float32),
        grid=(n_tiles,),
        in_specs=[
            pl.BlockSpec((None, 64, 81 * tb), lambda b: (b, 0, 0)),
            pl.BlockSpec(w1s.shape, lambda b: (0, 0)),
            pl.BlockSpec(w23.shape, lambda b: (0, 0)),
            pl.BlockSpec(whead.shape, lambda b: (0, 0)),
            pl.BlockSpec(biases.shape, lambda b: (0, 0)),
        ],
        out_specs=pl.BlockSpec((tb, _QPAD), lambda b: (b, 0)),
        compiler_params=pltpu.CompilerParams(
            dimension_semantics=("parallel",),
            vmem_limit_bytes=48 * 1024 * 1024),
    )(xs, w1s, w23, whead, biases)

    return out[:B, 1:1 + _A]


# speedup vs baseline: 1.6392x; 1.4247x over previous
import functools

import jax
import jax.numpy as jnp
from jax import lax
from jax.experimental import pallas as pl
from jax.experimental.pallas import tpu as pltpu

_QPAD = 128
_A = 6


def _round_up(x, m):
    return (x + m - 1) // m * m


def _fwd_kernel(x_ref, w1_ref, w23_ref, wh_ref, b_ref, q_ref, *,
                tb, c1, c2, c3, two_h):
    f32 = jnp.float32
    cdt = w1_ref.dtype

    cols = x_ref.shape[1]
    xv = x_ref[...]
    parts = [xv]
    for s in (tb, 9 * tb, 10 * tb):
        parts.append(jnp.concatenate(
            [xv[:, s:], jnp.zeros((64, s), cdt)], axis=1))
    parts.append(jnp.full((2, cols), 1, cdt))
    parts.append(jnp.zeros((14, cols), cdt))
    lhs1 = jnp.concatenate(parts, axis=0)
    z1t = lax.dot_general(w1_ref[...], lhs1, (((0,), (0,)), ((), ())),
                          preferred_element_type=f32)
    a1t = jnp.maximum(z1t, 0.0).astype(cdt)

    pieces = []
    for th in range(4):
        for tw in range(4):
            pieces.append(jnp.concatenate(
                [a1t[:, ((2 * i + th) * 9 + (2 * j + tw)) * tb:
                        ((2 * i + th) * 9 + (2 * j + tw)) * tb + tb]
                 for i in range(3) for j in range(3)], axis=1))
    lhs2 = jnp.concatenate(pieces, axis=0)
    z2 = lax.dot_general(lhs2, w23_ref[0:16 * c1, :],
                         (((0,), (0,)), ((), ())),
                         preferred_element_type=f32)
    x2 = jnp.maximum(z2 + b_ref[1:2, 0:c2], 0.0).astype(cdt)

    w3_off = 16 * c1
    zf = None
    for p in range(9):
        lhs = x2[p * tb:(p + 1) * tb, :]
        rhs = w23_ref[w3_off + p * c2:w3_off + (p + 1) * c2, :]
        c = jnp.dot(lhs, rhs, preferred_element_type=f32)
        zf = c if zf is None else zf + c
    feat = jnp.maximum(zf + b_ref[2:3, 0:c3], 0.0).astype(cdt)

    h = jnp.maximum(
        jnp.dot(feat, wh_ref[0:c3, 0:two_h], preferred_element_type=f32)
        + b_ref[3:4, 0:two_h], 0.0).astype(cdt)
    qpad = q_ref.shape[1]
    va = (jnp.dot(h, wh_ref[c3:c3 + two_h, 0:qpad],
                  preferred_element_type=f32)
          + b_ref[4:5, 0:qpad])

    value = va[:, 0:1]
    col = lax.broadcasted_iota(jnp.int32, va.shape, 1)
    adv_mask = (col >= 1) & (col < 1 + _A)
    adv_mean = jnp.sum(jnp.where(adv_mask, va, 0.0), axis=1,
                       keepdims=True) * (1.0 / _A)
    q_ref[...] = value + va - adv_mean


def kernel(w1, w23, whead, biases, state_nchw, *, batch_tile=128):
    cdt = w1.dtype
    B = state_nchw.shape[0]
    c1 = w1.shape[1]
    c2 = w23.shape[1]
    c3 = c2
    hidden = (whead.shape[0] - c3) // 2
    two_h = 2 * hidden

    tb = min(_round_up(B, 16), batch_tile)
    n_tiles = pl.cdiv(B, tb)
    G = n_tiles * tb

    w1s = (w1.reshape(2, 4, 2, 4, 4, c1)
             .transpose(0, 2, 4, 1, 3, 5).reshape(4 * 64, c1))
    b1 = biases[0, 0:c1]
    b1_hi = b1.astype(cdt)
    b1_lo = (b1 - b1_hi.astype(jnp.float32)).astype(cdt)
    w1s = jnp.concatenate(
        [w1s, b1_hi[None, :], b1_lo[None, :],
         jnp.zeros((14, c1), cdt)], axis=0)

    x = state_nchw
    if G != B:
        x = jnp.pad(x, ((0, G - B), (0, 0), (0, 0), (0, 0)))
    xs = x.reshape(n_tiles, tb, 4, 9, 4, 9, 4)
    xs = (xs.transpose(0, 2, 4, 6, 3, 5, 1)
            .reshape(n_tiles, 64, 81 * tb).astype(cdt))

    body = functools.partial(_fwd_kernel, tb=tb, c1=c1, c2=c2, c3=c3,
                             two_h=two_h)
    out = pl.pallas_call(
        body,
        out_shape=jax.ShapeDtypeStruct((G, _QPAD), jnp.float32),
        grid=(n_tiles,),
        in_specs=[
            pl.BlockSpec((None, 64, 81 * tb), lambda b: (b, 0, 0)),
            pl.BlockSpec(w1s.shape, lambda b: (0, 0)),
            pl.BlockSpec(w23.shape, lambda b: (0, 0)),
            pl.BlockSpec(whead.shape, lambda b: (0, 0)),
            pl.BlockSpec(biases.shape, lambda b: (0, 0)),
        ],
        out_specs=pl.BlockSpec((tb, _QPAD), lambda b: (b, 0)),
        compiler_params=pltpu.CompilerParams(
            dimension_semantics=("parallel",),
            vmem_limit_bytes=48 * 1024 * 1024),
    )(xs, w1s, w23, whead, biases)

    return out[:B, 1:1 + _A]


# speedup vs baseline: 1.6749x; 1.0218x over previous
import functools

import jax
import jax.numpy as jnp
from jax import lax
from jax.experimental import pallas as pl
from jax.experimental.pallas import tpu as pltpu

_QPAD = 128
_A = 6


def _round_up(x, m):
    return (x + m - 1) // m * m


def _fwd_kernel(x_ref, w1_ref, w23_ref, wh_ref, b_ref, q_ref, *,
                tb, c1, c2, c3, two_h):
    f32 = jnp.float32
    cdt = w1_ref.dtype

    cols = x_ref.shape[1]
    xv = x_ref[...]
    parts = [xv]
    for s in (tb, 9 * tb, 10 * tb):
        parts.append(jnp.concatenate(
            [xv[:, s:], jnp.zeros((64, s), cdt)], axis=1))
    parts.append(jnp.full((2, cols), 1, cdt))
    parts.append(jnp.zeros((14, cols), cdt))
    lhs1 = jnp.concatenate(parts, axis=0)
    z1t = lax.dot_general(w1_ref[...], lhs1, (((0,), (0,)), ((), ())),
                          preferred_element_type=f32)
    a1t = jnp.maximum(z1t, 0.0).astype(cdt)

    pieces = []
    for th in range(4):
        for tw in range(4):
            pieces.append(jnp.concatenate(
                [a1t[:, ((2 * i + th) * 9 + (2 * j + tw)) * tb:
                        ((2 * i + th) * 9 + (2 * j + tw)) * tb + tb]
                 for i in range(3) for j in range(3)], axis=1))
    lhs2 = jnp.concatenate(pieces, axis=0)
    z2 = lax.dot_general(lhs2, w23_ref[0:16 * c1, :],
                         (((0,), (0,)), ((), ())),
                         preferred_element_type=f32)
    x2 = jnp.maximum(z2 + b_ref[1:2, 0:c2], 0.0).astype(cdt)

    w3_off = 16 * c1
    zf = None
    for p in range(9):
        lhs = x2[p * tb:(p + 1) * tb, :]
        rhs = w23_ref[w3_off + p * c2:w3_off + (p + 1) * c2, :]
        c = jnp.dot(lhs, rhs, preferred_element_type=f32)
        zf = c if zf is None else zf + c
    feat = jnp.maximum(zf + b_ref[2:3, 0:c3], 0.0).astype(cdt)

    h = jnp.maximum(
        jnp.dot(feat, wh_ref[0:c3, 0:two_h], preferred_element_type=f32)
        + b_ref[3:4, 0:two_h], 0.0).astype(cdt)
    qpad = q_ref.shape[1]
    va = (jnp.dot(h, wh_ref[c3:c3 + two_h, 0:qpad],
                  preferred_element_type=f32)
          + b_ref[4:5, 0:qpad])

    value = va[:, 0:1]
    col = lax.broadcasted_iota(jnp.int32, va.shape, 1)
    adv_mask = (col >= 1) & (col < 1 + _A)
    adv_mean = jnp.sum(jnp.where(adv_mask, va, 0.0), axis=1,
                       keepdims=True) * (1.0 / _A)
    q_ref[...] = value + va - adv_mean


def kernel(w1, w23, whead, biases, state_nchw, *, batch_tile=256):
    cdt = w1.dtype
    B = state_nchw.shape[0]
    c1 = w1.shape[1]
    c2 = w23.shape[1]
    c3 = c2
    hidden = (whead.shape[0] - c3) // 2
    two_h = 2 * hidden

    tb = min(_round_up(B, 16), batch_tile)
    n_tiles = pl.cdiv(B, tb)
    G = n_tiles * tb

    w1s = (w1.reshape(2, 4, 2, 4, 4, c1)
             .transpose(0, 2, 4, 1, 3, 5).reshape(4 * 64, c1))
    b1 = biases[0, 0:c1]
    b1_hi = b1.astype(cdt)
    b1_lo = (b1 - b1_hi.astype(jnp.float32)).astype(cdt)
    w1s = jnp.concatenate(
        [w1s, b1_hi[None, :], b1_lo[None, :],
         jnp.zeros((14, c1), cdt)], axis=0)

    x = state_nchw
    if G != B:
        x = jnp.pad(x, ((0, G - B), (0, 0), (0, 0), (0, 0)))
    xs = x.reshape(n_tiles, tb, 4, 9, 4, 9, 4)
    xs = (xs.transpose(0, 2, 4, 6, 3, 5, 1)
            .reshape(n_tiles, 64, 81 * tb).astype(cdt))

    body = functools.partial(_fwd_kernel, tb=tb, c1=c1, c2=c2, c3=c3,
                             two_h=two_h)
    out = pl.pallas_call(
        body,
        out_shape=jax.ShapeDtypeStruct((G, _QPAD), jnp.float32),
        grid=(n_tiles,),
        in_specs=[
            pl.BlockSpec((None, 64, 81 * tb), lambda b: (b, 0, 0)),
            pl.BlockSpec(w1s.shape, lambda b: (0, 0)),
            pl.BlockSpec(w23.shape, lambda b: (0, 0)),
            pl.BlockSpec(whead.shape, lambda b: (0, 0)),
            pl.BlockSpec(biases.shape, lambda b: (0, 0)),
        ],
        out_specs=pl.BlockSpec((tb, _QPAD), lambda b: (b, 0)),
        compiler_params=pltpu.CompilerParams(
            dimension_semantics=("parallel",),
            vmem_limit_bytes=48 * 1024 * 1024),
    )(xs, w1s, w23, whead, biases)

    return out[:B, 1:1 + _A]


# speedup vs baseline: 3.3283x; 1.9871x over previous
import functools

import jax
import jax.numpy as jnp
from jax import lax
from jax.experimental import pallas as pl
from jax.experimental.pallas import tpu as pltpu

_QPAD = 128
_A = 6


def _round_up(x, m):
    return (x + m - 1) // m * m


def _fwd_kernel(x_ref, w1_ref, w23_ref, wh_ref, b_ref, q_ref, *,
                tb, c1, c2, c3, two_h):
    f32 = jnp.float32
    cdt = w1_ref.dtype

    ones2 = jnp.full((2, tb), 1, cdt)
    zer14 = jnp.zeros((14, tb), cdt)
    wv1 = w1_ref[...]
    cols_out = []
    for oh in range(8):
        for ow in range(8):
            runs = [x_ref[(4 * oh + kh) * 144 + 16 * ow:
                          (4 * oh + kh) * 144 + 16 * ow + 32, :]
                    for kh in range(8)]
            lhs_p = jnp.concatenate(runs + [ones2, zer14], axis=0)
            cols_out.append(lax.dot_general(
                wv1, lhs_p, (((0,), (0,)), ((), ())),
                preferred_element_type=f32))
    z1t = jnp.concatenate(cols_out, axis=1)
    a1t = jnp.maximum(z1t, 0.0).astype(cdt)

    pieces = []
    for th in range(4):
        for tw in range(4):
            pieces.append(jnp.concatenate(
                [a1t[:, ((2 * i + th) * 8 + (2 * j + tw)) * tb:
                        ((2 * i + th) * 8 + (2 * j + tw)) * tb + tb]
                 for i in range(3) for j in range(3)], axis=1))
    lhs2 = jnp.concatenate(pieces, axis=0)
    z2 = lax.dot_general(lhs2, w23_ref[0:16 * c1, :],
                         (((0,), (0,)), ((), ())),
                         preferred_element_type=f32)
    x2 = jnp.maximum(z2 + b_ref[1:2, 0:c2], 0.0).astype(cdt)

    w3_off = 16 * c1
    zf = None
    for p in range(9):
        lhs = x2[p * tb:(p + 1) * tb, :]
        rhs = w23_ref[w3_off + p * c2:w3_off + (p + 1) * c2, :]
        c = jnp.dot(lhs, rhs, preferred_element_type=f32)
        zf = c if zf is None else zf + c
    feat = jnp.maximum(zf + b_ref[2:3, 0:c3], 0.0).astype(cdt)

    h = jnp.maximum(
        jnp.dot(feat, wh_ref[0:c3, 0:two_h], preferred_element_type=f32)
        + b_ref[3:4, 0:two_h], 0.0).astype(cdt)
    qpad = q_ref.shape[1]
    va = (jnp.dot(h, wh_ref[c3:c3 + two_h, 0:qpad],
                  preferred_element_type=f32)
          + b_ref[4:5, 0:qpad])

    value = va[:, 0:1]
    col = lax.broadcasted_iota(jnp.int32, va.shape, 1)
    adv_mask = (col >= 1) & (col < 1 + _A)
    adv_mean = jnp.sum(jnp.where(adv_mask, va, 0.0), axis=1,
                       keepdims=True) * (1.0 / _A)
    q_ref[...] = value + va - adv_mean


def kernel(w1, w23, whead, biases, state_nchw, *, batch_tile=256):
    cdt = w1.dtype
    B = state_nchw.shape[0]
    c1 = w1.shape[1]
    c2 = w23.shape[1]
    c3 = c2
    hidden = (whead.shape[0] - c3) // 2
    two_h = 2 * hidden

    tb = min(_round_up(B, 16), batch_tile)
    n_tiles = pl.cdiv(B, tb)
    G = n_tiles * tb

    b1 = biases[0, 0:c1]
    b1_hi = b1.astype(cdt)
    b1_lo = (b1 - b1_hi.astype(jnp.float32)).astype(cdt)
    w1s = jnp.concatenate(
        [w1, b1_hi[None, :], b1_lo[None, :],
         jnp.zeros((14, c1), cdt)], axis=0)

    x = state_nchw
    if G != B:
        x = jnp.pad(x, ((0, G - B), (0, 0), (0, 0), (0, 0)))
    xs = x.transpose(2, 3, 1, 0).reshape(36 * 36 * 4, G).astype(cdt)

    body = functools.partial(_fwd_kernel, tb=tb, c1=c1, c2=c2, c3=c3,
                             two_h=two_h)
    out = pl.pallas_call(
        body,
        out_shape=jax.ShapeDtypeStruct((G, _QPAD), jnp.float32),
        grid=(n_tiles,),
        in_specs=[
            pl.BlockSpec((36 * 36 * 4, tb), lambda b: (0, b)),
            pl.BlockSpec(w1s.shape, lambda b: (0, 0)),
            pl.BlockSpec(w23.shape, lambda b: (0, 0)),
            pl.BlockSpec(whead.shape, lambda b: (0, 0)),
            pl.BlockSpec(biases.shape, lambda b: (0, 0)),
        ],
        out_specs=pl.BlockSpec((tb, _QPAD), lambda b: (b, 0)),
        compiler_params=pltpu.CompilerParams(
            dimension_semantics=("parallel",),
            vmem_limit_bytes=48 * 1024 * 1024),
    )(xs, w1s, w23, whead, biases)

    return out[:B, 1:1 + _A]
